# baseline (speedup 1.0000x reference)
import jax
import jax.numpy as jnp
from jax import lax
from jax.experimental import pallas as pl
from jax.experimental.pallas import tpu as pltpu
from jax.experimental.pallas import tpu_sc as plsc

DIM = 64
BATCH = 16384
ENT = 1000000
L = 16
NC = 2
NS = 16
NW = NC * NS
B_W = BATCH // NW
P_W = B_W // 2
CHUNK = 128
NCOL = ENT // 128
COLS_PER_W = (NCOL + NW - 1) // NW + 1
EPAD = (NCOL + 1) * 64


def _transpose_body(entT_hbm, entp_hbm, tbuf, obuf, ttail, otail):
    wid = lax.axis_index("s") * NC + lax.axis_index("c")
    lane = lax.iota(jnp.int32, L)
    rows = [(m * L + lane) >> 1 for m in range(8)]
    colb = [((m * L + lane) & 1) * DIM for m in range(8)]

    def one(i, carry):
        c = wid + NW * i

        @pl.when(c < NCOL)
        def _():
            pltpu.sync_copy(entT_hbm.at[:, pl.ds(c * 128, 128)], tbuf)

            def drow(d, cc):
                for m in range(8):
                    v = tbuf[d, pl.ds(m * L, L)]
                    plsc.store_scatter(obuf, [rows[m], colb[m] + d], v)
                return cc

            lax.fori_loop(0, DIM, drow, 0)
            pltpu.sync_copy(obuf, entp_hbm.at[pl.ds(c * 64, 64), :])

        return carry

    lax.fori_loop(0, COLS_PER_W, one, 0)

    @pl.when(wid == NW - 1)
    def _():
        pltpu.sync_copy(entT_hbm.at[:, pl.ds(NCOL * 128, ENT - NCOL * 128)],
                        ttail)

        def drow2(d, cc):
            for m in range(4):
                v = ttail[d, pl.ds(m * L, L)]
                plsc.store_scatter(otail, [rows[m], colb[m] + d], v)
            return cc

        lax.fori_loop(0, DIM, drow2, 0)
        pltpu.sync_copy(otail, entp_hbm.at[pl.ds(NCOL * 64, 32), :])


def _gather_body(entp_hbm, rel_hbm, hp_hbm, ho_hbm, rp_hbm, ro_hbm,
                 tp_hbm, to_hbm, psum_hbm,
                 hpi, hoi, rpi, roi, tpi, toi, hbuf, rbuf, tbuf, pbuf, sem):
    wid = lax.axis_index("s") * NC + lax.axis_index("c")
    base = wid * B_W

    for p in range(2):
        pb = base + p * P_W
        pltpu.sync_copy(hp_hbm.at[pl.ds(pb, P_W)], hpi)
        pltpu.sync_copy(ho_hbm.at[pl.ds(pb, P_W)], hoi)
        pltpu.sync_copy(rp_hbm.at[pl.ds(pb, P_W)], rpi)
        pltpu.sync_copy(ro_hbm.at[pl.ds(pb, P_W)], roi)
        pltpu.sync_copy(tp_hbm.at[pl.ds(pb, P_W)], tpi)
        pltpu.sync_copy(to_hbm.at[pl.ds(pb, P_W)], toi)

        copies = []
        for j in range(P_W // CHUNK):
            sl = pl.ds(j * CHUNK, CHUNK)
            copies.append(pltpu.async_copy(entp_hbm.at[hpi.at[sl]], hbuf.at[sl], sem))
            copies.append(pltpu.async_copy(rel_hbm.at[rpi.at[sl]], rbuf.at[sl], sem))
            copies.append(pltpu.async_copy(entp_hbm.at[tpi.at[sl]], tbuf.at[sl], sem))
        for c in copies:
            c.wait()

        def group(g, carry):
            hov = hoi[pl.ds(g * L, L)]
            rov = roi[pl.ds(g * L, L)]
            tov = toi[pl.ds(g * L, L)]
            for j in range(L):
                i = g * L + j
                ho = hov[j]
                ro = rov[j]
                to = tov[j]
                acc = jnp.zeros((L,), jnp.float32)
                for k in range(DIM // L):
                    o = k * L
                    diff = (hbuf[i, pl.ds(ho + o, L)]
                            + rbuf[i, pl.ds(ro + o, L)]
                            - tbuf[i, pl.ds(to + o, L)])
                    acc = acc + diff * diff
                ii = p * P_W + i
                pbuf[ii >> 3, pl.ds((j & 7) * L, L)] = acc
            return carry

        lax.fori_loop(0, P_W // L, group, 0)

    pltpu.sync_copy(pbuf, psum_hbm.at[pl.ds(wid * (B_W // 8), B_W // 8)])


def _tc_norm_body(p_ref, o_ref):
    p = p_ref[...]
    lane_grp = lax.broadcasted_iota(jnp.int32, (128, 8), 0) // L
    out_grp = lax.broadcasted_iota(jnp.int32, (128, 8), 1)
    sel = (lane_grp == out_grp).astype(jnp.float32)
    o_ref[...] = jnp.sqrt(
        lax.dot_general(p, sel, (((1,), (0,)), ((), ())),
                        precision=lax.Precision.HIGHEST,
                        preferred_element_type=jnp.float32))


def kernel(ent_emb, rel_emb, h, r, t):
    h = h.astype(jnp.int32)
    r = r.astype(jnp.int32)
    t = t.astype(jnp.int32)
    ent_T = ent_emb.T
    rel2 = rel_emb.reshape(rel_emb.shape[0] // 2, 2 * DIM)
    hp, ho = h >> 1, (h & 1) << 6
    rp, ro = r >> 1, (r & 1) << 6
    tp, to = t >> 1, (t & 1) << 6
    mesh = plsc.VectorSubcoreMesh(core_axis_name="c", subcore_axis_name="s")

    transpose = pl.kernel(
        _transpose_body,
        mesh=mesh,
        compiler_params=pltpu.CompilerParams(needs_layout_passes=False),
        out_type=jax.ShapeDtypeStruct((EPAD, 2 * DIM), jnp.float32),
        scratch_types=[
            pltpu.VMEM((DIM, 128), jnp.float32),
            pltpu.VMEM((64, 128), jnp.float32),
            pltpu.VMEM((DIM, 64), jnp.float32),
            pltpu.VMEM((32, 128), jnp.float32),
        ],
    )
    entp = transpose(ent_T)

    gather_partials = pl.kernel(
        _gather_body,
        mesh=mesh,
        out_type=jax.ShapeDtypeStruct((BATCH // 8, 8 * L), jnp.float32),
        scratch_types=[
            pltpu.VMEM((P_W,), jnp.int32),
            pltpu.VMEM((P_W,), jnp.int32),
            pltpu.VMEM((P_W,), jnp.int32),
            pltpu.VMEM((P_W,), jnp.int32),
            pltpu.VMEM((P_W,), jnp.int32),
            pltpu.VMEM((P_W,), jnp.int32),
            pltpu.VMEM((P_W, 2 * DIM), jnp.float32),
            pltpu.VMEM((P_W, 2 * DIM), jnp.float32),
            pltpu.VMEM((P_W, 2 * DIM), jnp.float32),
            pltpu.VMEM((B_W // 8, 8 * L), jnp.float32),
            pltpu.SemaphoreType.DMA,
        ],
    )
    psums = gather_partials(entp, rel2, hp, ho, rp, ro, tp, to)

    norms = pl.pallas_call(
        _tc_norm_body,
        out_shape=jax.ShapeDtypeStruct((BATCH // 8, 8), jnp.float32),
    )(psums)
    return norms.reshape(BATCH)


# speedup vs baseline: 4.1932x; 4.1932x over previous
import jax
import jax.numpy as jnp
from jax import lax
from jax.experimental import pallas as pl
from jax.experimental.pallas import tpu as pltpu
from jax.experimental.pallas import tpu_sc as plsc

DIM = 64
BATCH = 16384
L = 16
NC = 2
NS = 16
NW = NC * NS
B_W = BATCH // NW
P_W = 128
NPASS = B_W // P_W


def _tec_body(ent_hbm, rel_hbm, h_hbm, r_hbm, t_hbm, psum_hbm,
              hidx, ridx, tidx, hbuf, rbuf, tbuf, pbuf, sem):
    wid = lax.axis_index("s") * NC + lax.axis_index("c")
    base = wid * B_W

    def one_pass(p, carry):
        pb = base + p * P_W
        pltpu.sync_copy(h_hbm.at[pl.ds(pb, P_W)], hidx)
        pltpu.sync_copy(r_hbm.at[pl.ds(pb, P_W)], ridx)
        pltpu.sync_copy(t_hbm.at[pl.ds(pb, P_W)], tidx)

        def fire(g, carry2):
            hv = hidx[pl.ds(g * L, L)]
            rv = ridx[pl.ds(g * L, L)]
            tv = tidx[pl.ds(g * L, L)]
            for j in range(L):
                e = g * L + j
                pltpu.async_copy(ent_hbm.at[hv[j]], hbuf.at[e], sem)
                pltpu.async_copy(rel_hbm.at[rv[j]], rbuf.at[e], sem)
                pltpu.async_copy(ent_hbm.at[tv[j]], tbuf.at[e], sem)
            return carry2

        lax.fori_loop(0, P_W // L, fire, 0)

        pltpu.make_async_copy(ent_hbm.at[pl.ds(0, P_W)], hbuf, sem).wait()
        pltpu.make_async_copy(ent_hbm.at[pl.ds(0, P_W)], tbuf, sem).wait()
        pltpu.make_async_copy(rel_hbm.at[pl.ds(0, P_W)], rbuf, sem).wait()

        def group(g, carry2):
            for j in range(L):
                e = g * L + j
                acc = jnp.zeros((L,), jnp.float32)
                for k in range(DIM // L):
                    sl = pl.ds(k * L, L)
                    diff = hbuf[e, sl] + rbuf[e, sl] - tbuf[e, sl]
                    acc = acc + diff * diff
                pbuf[p * (P_W // 8) + (e >> 3), pl.ds((j & 7) * L, L)] = acc
            return carry2

        lax.fori_loop(0, P_W // L, group, 0)
        return carry

    lax.fori_loop(0, NPASS, one_pass, 0)

    pltpu.sync_copy(pbuf, psum_hbm.at[pl.ds(wid * (B_W // 8), B_W // 8)])


def _tc_norm_body(p_ref, o_ref):
    p = p_ref[...]
    lane_grp = lax.broadcasted_iota(jnp.int32, (128, 8), 0) // L
    out_grp = lax.broadcasted_iota(jnp.int32, (128, 8), 1)
    sel = (lane_grp == out_grp).astype(jnp.float32)
    o_ref[...] = jnp.sqrt(
        lax.dot_general(p, sel, (((1,), (0,)), ((), ())),
                        precision=lax.Precision.HIGHEST,
                        preferred_element_type=jnp.float32))


def kernel(ent_emb, rel_emb, h, r, t):
    h = h.astype(jnp.int32)
    r = r.astype(jnp.int32)
    t = t.astype(jnp.int32)
    mesh = plsc.VectorSubcoreMesh(core_axis_name="c", subcore_axis_name="s")
    gather_partials = pl.kernel(
        _tec_body,
        mesh=mesh,
        out_type=jax.ShapeDtypeStruct((BATCH // 8, 8 * L), jnp.float32),
        scratch_types=[
            pltpu.VMEM((P_W,), jnp.int32),
            pltpu.VMEM((P_W,), jnp.int32),
            pltpu.VMEM((P_W,), jnp.int32),
            pltpu.VMEM((P_W, DIM), jnp.float32),
            pltpu.VMEM((P_W, DIM), jnp.float32),
            pltpu.VMEM((P_W, DIM), jnp.float32),
            pltpu.VMEM((B_W // 8, 8 * L), jnp.float32),
            pltpu.SemaphoreType.DMA,
        ],
    )
    psums = gather_partials(ent_emb, rel_emb, h, r, t)
    norms = pl.pallas_call(
        _tc_norm_body,
        out_shape=jax.ShapeDtypeStruct((BATCH // 8, 8), jnp.float32),
    )(psums)
    return norms.reshape(BATCH)


# speedup vs baseline: 6.1316x; 1.4623x over previous
import jax
import jax.numpy as jnp
from jax import lax
from jax.experimental import pallas as pl
from jax.experimental.pallas import tpu as pltpu
from jax.experimental.pallas import tpu_sc as plsc

DIM = 64
BATCH = 16384
L = 16
NC = 2
NS = 16
NW = NC * NS
B_W = BATCH // NW
P_W = 128
NPASS = B_W // P_W


def _tec_body(ent_hbm, rel_hbm, h_hbm, r_hbm, t_hbm, psum_hbm,
              hidx, ridx, tidx, hbuf, rbuf, tbuf, pbuf, sem):
    wid = lax.axis_index("s") * NC + lax.axis_index("c")
    base = wid * B_W

    def one_pass(p, carry):
        pb = base + p * P_W
        pltpu.sync_copy(h_hbm.at[pl.ds(pb, P_W)], hidx)
        pltpu.sync_copy(r_hbm.at[pl.ds(pb, P_W)], ridx)
        pltpu.sync_copy(t_hbm.at[pl.ds(pb, P_W)], tidx)

        def fire(g, carry2):
            hv = hidx[pl.ds(g * L, L)]
            rv = ridx[pl.ds(g * L, L)]
            tv = tidx[pl.ds(g * L, L)]
            for j in range(L):
                e = g * L + j
                pltpu.async_copy(ent_hbm.at[hv[j]], hbuf.at[e], sem)
                pltpu.async_copy(rel_hbm.at[rv[j]], rbuf.at[e], sem)
                pltpu.async_copy(ent_hbm.at[tv[j]], tbuf.at[e], sem)
            return carry2

        lax.fori_loop(0, P_W // L, fire, 0)

        pltpu.make_async_copy(ent_hbm.at[pl.ds(0, P_W)], hbuf, sem).wait()
        pltpu.make_async_copy(ent_hbm.at[pl.ds(0, P_W)], tbuf, sem).wait()
        pltpu.make_async_copy(rel_hbm.at[pl.ds(0, P_W)], rbuf, sem).wait()

        def group(g, carry2):
            for j in range(L):
                e = g * L + j
                acc = jnp.zeros((L,), jnp.float32)
                for k in range(DIM // L):
                    sl = pl.ds(k * L, L)
                    diff = hbuf[e, sl] + rbuf[e, sl] - tbuf[e, sl]
                    acc = acc + diff * diff
                pbuf[p * (P_W // 8) + (e >> 3), pl.ds((j & 7) * L, L)] = acc
            return carry2

        lax.fori_loop(0, P_W // L, group, 0)
        return carry

    lax.fori_loop(0, NPASS, one_pass, 0)

    pltpu.sync_copy(pbuf, psum_hbm.at[pl.ds(wid * (B_W // 8), B_W // 8)])


def _tc_norm_body(p_ref, o_ref):
    p = p_ref[...]
    lane_grp = lax.broadcasted_iota(jnp.int32, (128, 8), 0) // L
    out_grp = lax.broadcasted_iota(jnp.int32, (128, 8), 1)
    sel = (lane_grp == out_grp).astype(jnp.float32)
    o_ref[...] = jnp.sqrt(
        lax.dot_general(p, sel, (((1,), (0,)), ((), ())),
                        precision=lax.Precision.HIGHEST,
                        preferred_element_type=jnp.float32))


def kernel(ent_emb, rel_emb, h, r, t):
    h = h.astype(jnp.int32)
    r = r.astype(jnp.int32)
    t = t.astype(jnp.int32)
    zupd = jnp.broadcast_to((r[:1].astype(jnp.float32) * 0.0)[:, None], (1, DIM))
    ent_emb = ent_emb.at[h[:1]].add(zupd)
    mesh = plsc.VectorSubcoreMesh(core_axis_name="c", subcore_axis_name="s")
    gather_partials = pl.kernel(
        _tec_body,
        mesh=mesh,
        out_type=jax.ShapeDtypeStruct((BATCH // 8, 8 * L), jnp.float32),
        scratch_types=[
            pltpu.VMEM((P_W,), jnp.int32),
            pltpu.VMEM((P_W,), jnp.int32),
            pltpu.VMEM((P_W,), jnp.int32),
            pltpu.VMEM((P_W, DIM), jnp.float32),
            pltpu.VMEM((P_W, DIM), jnp.float32),
            pltpu.VMEM((P_W, DIM), jnp.float32),
            pltpu.VMEM((B_W // 8, 8 * L), jnp.float32),
            pltpu.SemaphoreType.DMA,
        ],
    )
    psums = gather_partials(ent_emb, rel_emb, h, r, t)
    norms = pl.pallas_call(
        _tc_norm_body,
        out_shape=jax.ShapeDtypeStruct((BATCH // 8, 8), jnp.float32),
    )(psums)
    return norms.reshape(BATCH)


# speedup vs baseline: 6.2171x; 1.0140x over previous
import jax
import jax.numpy as jnp
from jax import lax
from jax.experimental import pallas as pl
from jax.experimental.pallas import tpu as pltpu
from jax.experimental.pallas import tpu_sc as plsc

DIM = 64
BATCH = 16384
L = 16
NC = 2
NS = 16
NW = NC * NS
B_W = BATCH // NW
P_W = 256
NPASS = B_W // P_W


def _tec_body(ent_hbm, rel_hbm, h_hbm, r_hbm, t_hbm, psum_hbm,
              hidx, ridx, tidx, hbuf, rbuf, tbuf, pbuf, sem):
    wid = lax.axis_index("s") * NC + lax.axis_index("c")
    base = wid * B_W

    def one_pass(p, carry):
        pb = base + p * P_W
        pltpu.sync_copy(h_hbm.at[pl.ds(pb, P_W)], hidx)
        pltpu.sync_copy(r_hbm.at[pl.ds(pb, P_W)], ridx)
        pltpu.sync_copy(t_hbm.at[pl.ds(pb, P_W)], tidx)

        def fire(g, carry2):
            hv = hidx[pl.ds(g * L, L)]
            rv = ridx[pl.ds(g * L, L)]
            tv = tidx[pl.ds(g * L, L)]
            for j in range(L):
                e = g * L + j
                pltpu.async_copy(ent_hbm.at[hv[j]], hbuf.at[e], sem)
                pltpu.async_copy(rel_hbm.at[rv[j]], rbuf.at[e], sem)
                pltpu.async_copy(ent_hbm.at[tv[j]], tbuf.at[e], sem)
            return carry2

        lax.fori_loop(0, P_W // L, fire, 0)

        pltpu.make_async_copy(ent_hbm.at[pl.ds(0, P_W)], hbuf, sem).wait()
        pltpu.make_async_copy(ent_hbm.at[pl.ds(0, P_W)], tbuf, sem).wait()
        pltpu.make_async_copy(rel_hbm.at[pl.ds(0, P_W)], rbuf, sem).wait()

        def group(g, carry2):
            for j in range(L):
                e = g * L + j
                acc = jnp.zeros((L,), jnp.float32)
                for k in range(DIM // L):
                    sl = pl.ds(k * L, L)
                    diff = hbuf[e, sl] + rbuf[e, sl] - tbuf[e, sl]
                    acc = acc + diff * diff
                pbuf[p * (P_W // 8) + (e >> 3), pl.ds((j & 7) * L, L)] = acc
            return carry2

        lax.fori_loop(0, P_W // L, group, 0)
        return carry

    lax.fori_loop(0, NPASS, one_pass, 0)

    pltpu.sync_copy(pbuf, psum_hbm.at[pl.ds(wid * (B_W // 8), B_W // 8)])


def _tc_norm_body(p_ref, o_ref):
    p = p_ref[...]
    lane_grp = lax.broadcasted_iota(jnp.int32, (128, 8), 0) // L
    out_grp = lax.broadcasted_iota(jnp.int32, (128, 8), 1)
    sel = (lane_grp == out_grp).astype(jnp.float32)
    o_ref[...] = jnp.sqrt(
        lax.dot_general(p, sel, (((1,), (0,)), ((), ())),
                        precision=lax.Precision.HIGHEST,
                        preferred_element_type=jnp.float32))


def kernel(ent_emb, rel_emb, h, r, t):
    h = h.astype(jnp.int32)
    r = r.astype(jnp.int32)
    t = t.astype(jnp.int32)
    zupd = jnp.broadcast_to((r[:1].astype(jnp.float32) * 0.0)[:, None], (1, DIM))
    ent_emb = ent_emb.at[h[:1]].add(zupd)
    mesh = plsc.VectorSubcoreMesh(core_axis_name="c", subcore_axis_name="s")
    gather_partials = pl.kernel(
        _tec_body,
        mesh=mesh,
        out_type=jax.ShapeDtypeStruct((BATCH // 8, 8 * L), jnp.float32),
        scratch_types=[
            pltpu.VMEM((P_W,), jnp.int32),
            pltpu.VMEM((P_W,), jnp.int32),
            pltpu.VMEM((P_W,), jnp.int32),
            pltpu.VMEM((P_W, DIM), jnp.float32),
            pltpu.VMEM((P_W, DIM), jnp.float32),
            pltpu.VMEM((P_W, DIM), jnp.float32),
            pltpu.VMEM((B_W // 8, 8 * L), jnp.float32),
            pltpu.SemaphoreType.DMA,
        ],
    )
    psums = gather_partials(ent_emb, rel_emb, h, r, t)
    norms = pl.pallas_call(
        _tc_norm_body,
        out_shape=jax.ShapeDtypeStruct((BATCH // 8, 8), jnp.float32),
    )(psums)
    return norms.reshape(BATCH)


# speedup vs baseline: 6.3245x; 1.0173x over previous
import jax
import jax.numpy as jnp
from jax import lax
from jax.experimental import pallas as pl
from jax.experimental.pallas import tpu as pltpu
from jax.experimental.pallas import tpu_sc as plsc

DIM = 64
BATCH = 16384
L = 16
NC = 2
NS = 16
NW = NC * NS
B_W = BATCH // NW
P_W = 128
NPASS = B_W // P_W


def _tec_body(ent_hbm, rel_hbm, h_hbm, r_hbm, t_hbm, psum_hbm,
              hidx, ridx, tidx,
              hbuf0, rbuf0, tbuf0, hbuf1, rbuf1, tbuf1, pbuf, sem0, sem1):
    wid = lax.axis_index("s") * NC + lax.axis_index("c")
    base = wid * B_W

    pltpu.sync_copy(h_hbm.at[pl.ds(base, B_W)], hidx)
    pltpu.sync_copy(r_hbm.at[pl.ds(base, B_W)], ridx)
    pltpu.sync_copy(t_hbm.at[pl.ds(base, B_W)], tidx)

    bufs = ((hbuf0, rbuf0, tbuf0, sem0), (hbuf1, rbuf1, tbuf1, sem1))

    def fire(p, parity):
        hbuf, rbuf, tbuf, sem = bufs[parity]

        def fire_g(g, carry2):
            off = p * P_W + g * L
            hv = hidx[pl.ds(off, L)]
            rv = ridx[pl.ds(off, L)]
            tv = tidx[pl.ds(off, L)]
            for j in range(L):
                e = g * L + j
                pltpu.async_copy(ent_hbm.at[hv[j]], hbuf.at[e], sem)
                pltpu.async_copy(rel_hbm.at[rv[j]], rbuf.at[e], sem)
                pltpu.async_copy(ent_hbm.at[tv[j]], tbuf.at[e], sem)
            return carry2

        lax.fori_loop(0, P_W // L, fire_g, 0)

    def drain_compute(p, parity):
        hbuf, rbuf, tbuf, sem = bufs[parity]
        pltpu.make_async_copy(ent_hbm.at[pl.ds(0, P_W)], hbuf, sem).wait()
        pltpu.make_async_copy(ent_hbm.at[pl.ds(0, P_W)], tbuf, sem).wait()
        pltpu.make_async_copy(rel_hbm.at[pl.ds(0, P_W)], rbuf, sem).wait()

        def group(g, carry2):
            for j in range(L):
                e = g * L + j
                acc = jnp.zeros((L,), jnp.float32)
                for k in range(DIM // L):
                    sl = pl.ds(k * L, L)
                    diff = hbuf[e, sl] + rbuf[e, sl] - tbuf[e, sl]
                    acc = acc + diff * diff
                pbuf[p * (P_W // 8) + (e >> 3), pl.ds((j & 7) * L, L)] = acc
            return carry2

        lax.fori_loop(0, P_W // L, group, 0)

    fire(0, 0)

    def body(i, carry):
        p0 = 2 * i
        fire(p0 + 1, 1)
        drain_compute(p0, 0)

        @pl.when(p0 + 2 < NPASS)
        def _():
            fire(p0 + 2, 0)

        drain_compute(p0 + 1, 1)
        return carry

    lax.fori_loop(0, NPASS // 2, body, 0)

    pltpu.sync_copy(pbuf, psum_hbm.at[pl.ds(wid * (B_W // 8), B_W // 8)])


def _tc_norm_body(p_ref, o_ref):
    p = p_ref[...]
    lane_grp = lax.broadcasted_iota(jnp.int32, (128, 8), 0) // L
    out_grp = lax.broadcasted_iota(jnp.int32, (128, 8), 1)
    sel = (lane_grp == out_grp).astype(jnp.float32)
    o_ref[...] = jnp.sqrt(
        lax.dot_general(p, sel, (((1,), (0,)), ((), ())),
                        precision=lax.Precision.HIGHEST,
                        preferred_element_type=jnp.float32))


def kernel(ent_emb, rel_emb, h, r, t):
    h = h.astype(jnp.int32)
    r = r.astype(jnp.int32)
    t = t.astype(jnp.int32)
    zupd = jnp.broadcast_to((r[:1].astype(jnp.float32) * 0.0)[:, None], (1, DIM))
    ent_emb = ent_emb.at[h[:1]].add(zupd)
    mesh = plsc.VectorSubcoreMesh(core_axis_name="c", subcore_axis_name="s")
    gather_partials = pl.kernel(
        _tec_body,
        mesh=mesh,
        out_type=jax.ShapeDtypeStruct((BATCH // 8, 8 * L), jnp.float32),
        scratch_types=[
            pltpu.VMEM((B_W,), jnp.int32),
            pltpu.VMEM((B_W,), jnp.int32),
            pltpu.VMEM((B_W,), jnp.int32),
            pltpu.VMEM((P_W, DIM), jnp.float32),
            pltpu.VMEM((P_W, DIM), jnp.float32),
            pltpu.VMEM((P_W, DIM), jnp.float32),
            pltpu.VMEM((P_W, DIM), jnp.float32),
            pltpu.VMEM((P_W, DIM), jnp.float32),
            pltpu.VMEM((P_W, DIM), jnp.float32),
            pltpu.VMEM((B_W // 8, 8 * L), jnp.float32),
            pltpu.SemaphoreType.DMA,
            pltpu.SemaphoreType.DMA,
        ],
    )
    psums = gather_partials(ent_emb, rel_emb, h, r, t)
    norms = pl.pallas_call(
        _tc_norm_body,
        out_shape=jax.ShapeDtypeStruct((BATCH // 8, 8), jnp.float32),
    )(psums)
    return norms.reshape(BATCH)
